# Initial kernel scaffold; baseline (speedup 1.0000x reference)
import jax
import jax.numpy as jnp
from jax.experimental import pallas as pl


def kernel(x, edge_index, W1, b1, W2, b2):
    raise NotImplementedError("write your pallas kernel here")


# speedup vs baseline: 3.8819x; 3.8819x over previous
import functools

import jax
import jax.numpy as jnp
from jax import lax
from jax.experimental import pallas as pl
from jax.experimental.pallas import tpu as pltpu
from jax.experimental.pallas import tpu_sc as plsc

N_NODES = 10000
N_PAD = 10240
CH = 128
NC = 2
NS = 16
NW = NC * NS
NCH1 = 80
EPT1 = NCH1 * CH
NCH2 = 158
EPT2 = NCH2 * CH
ROW_BLK = 128

_mesh = plsc.VectorSubcoreMesh(core_axis_name="c", subcore_axis_name="s")
_sc_params = pltpu.CompilerParams(use_tc_tiling_on_sc=False)


@functools.partial(
    pl.kernel,
    out_type=jax.ShapeDtypeStruct((NC, 2 * N_PAD), jnp.float32),
    mesh=_mesh,
    compiler_params=_sc_params,
    scratch_types=[
        pltpu.VMEM((NCH2, CH), jnp.int32),
        pltpu.VMEM((CH,), jnp.float32),
        pltpu.VMEM_SHARED((2 * N_PAD,), jnp.float32),
    ],
)
def _deg_kernel(cidx_hbm, ones_hbm, zeros_hbm, out_hbm, idx_all, ones_v, acc):
    cc = lax.axis_index("c")
    ss = lax.axis_index("s")
    wid = ss * NC + cc
    span = 2 * N_PAD // NS
    pltpu.sync_copy(zeros_hbm, acc.at[pl.ds(ss * span, span)])
    pltpu.sync_copy(cidx_hbm.at[wid], idx_all)
    pltpu.sync_copy(ones_hbm, ones_v)
    plsc.subcore_barrier()

    @pl.loop(0, NCH2)
    def _(t):
        pltpu.sync_copy(ones_v, acc.at[idx_all.at[t]], add=True)

    plsc.subcore_barrier()
    pltpu.sync_copy(acc.at[pl.ds(ss * span, span)],
                    out_hbm.at[cc, pl.ds(ss * span, span)])


def _make_agg(phases):
    @functools.partial(
        pl.kernel,
        out_type=jax.ShapeDtypeStruct((phases, NC, N_PAD, 64), jnp.float32),
        mesh=_mesh,
        compiler_params=_sc_params,
        scratch_types=[
            pltpu.VMEM((NCH1, CH), jnp.int32),
            pltpu.VMEM((NCH1, CH), jnp.int32),
            pltpu.VMEM((CH, 64), jnp.float32),
            pltpu.VMEM((CH, 64), jnp.float32),
            pltpu.VMEM_SHARED((N_PAD, 64), jnp.float32),
            pltpu.SemaphoreType.DMA,
            pltpu.SemaphoreType.DMA,
        ],
    )
    def agg(table_hbm, sidx_hbm, didx_hbm, zeros_hbm, out_hbm,
            sidx, didx, rows0, rows1, acc, gsem0, gsem1):
        cc = lax.axis_index("c")
        ss = lax.axis_index("s")
        wid = ss * NC + cc
        rpt = N_PAD // NS
        pltpu.sync_copy(sidx_hbm.at[wid], sidx)
        pltpu.sync_copy(didx_hbm.at[wid], didx)

        for ph in range(phases):
            pltpu.sync_copy(zeros_hbm, acc.at[pl.ds(ss * rpt, rpt)])
            plsc.subcore_barrier()
            table = table_hbm.at[ph]

            pltpu.async_copy(table.at[sidx.at[0]], rows0, gsem0)

            @pl.loop(0, NCH1 // 2)
            def _(p):
                t0 = 2 * p
                pltpu.make_async_copy(table.at[sidx.at[t0]], rows0, gsem0).wait()
                pltpu.async_copy(table.at[sidx.at[t0 + 1]], rows1, gsem1)
                pltpu.sync_copy(rows0, acc.at[didx.at[t0]], add=True)
                pltpu.make_async_copy(table.at[sidx.at[t0 + 1]], rows1, gsem1).wait()

                @pl.when(t0 + 2 < NCH1)
                def _():
                    pltpu.async_copy(table.at[sidx.at[t0 + 2]], rows0, gsem0)

                pltpu.sync_copy(rows1, acc.at[didx.at[t0 + 1]], add=True)

            plsc.subcore_barrier()
            pltpu.sync_copy(acc.at[pl.ds(ss * rpt, rpt)],
                            out_hbm.at[ph, cc, pl.ds(ss * rpt, rpt)])

    return agg


_agg2 = _make_agg(2)
_agg1 = _make_agg(1)


def _scale_body(x_ref, ds_ref, o_ref):
    ns = lax.rsqrt(jnp.maximum(ds_ref[...], 1.0))
    o_ref[...] = x_ref[...] * ns


def _mid_body(lo_ref, hi_ref, dd_ref, ds_ref, w1lo_ref, w1hi_ref, b1_ref,
              w2_ref, o_ref):
    nd = lax.rsqrt(jnp.maximum(dd_ref[...], 1.0))
    ns = lax.rsqrt(jnp.maximum(ds_ref[...], 1.0))
    agg_lo = (lo_ref[0] + lo_ref[1]) * nd
    agg_hi = (hi_ref[0] + hi_ref[1]) * nd
    z1 = (jnp.dot(agg_lo, w1lo_ref[...], preferred_element_type=jnp.float32)
          + jnp.dot(agg_hi, w1hi_ref[...], preferred_element_type=jnp.float32)
          + b1_ref[...])
    o_ref[...] = jnp.dot(z1 * ns, w2_ref[...], preferred_element_type=jnp.float32)


def _fin_body(p_ref, dd_ref, b2_ref, o_ref):
    nd = lax.rsqrt(jnp.maximum(dd_ref[...], 1.0))
    o_ref[...] = (p_ref[0] + p_ref[1]) * nd + b2_ref[...]


def _col_spec():
    return pl.BlockSpec((ROW_BLK, 1), lambda i: (i, 0))


def _scale(x_pad, degs_col):
    return pl.pallas_call(
        _scale_body,
        grid=(N_PAD // ROW_BLK,),
        in_specs=[pl.BlockSpec((ROW_BLK, 128), lambda i: (i, 0)), _col_spec()],
        out_specs=pl.BlockSpec((ROW_BLK, 128), lambda i: (i, 0)),
        out_shape=jax.ShapeDtypeStruct((N_PAD, 128), jnp.float32),
    )(x_pad, degs_col)


def _mid(p1, degd_col, degs_col, W1, b1, W2):
    return pl.pallas_call(
        _mid_body,
        grid=(N_PAD // ROW_BLK,),
        in_specs=[
            pl.BlockSpec((NC, ROW_BLK, 64), lambda i: (0, i, 0)),
            pl.BlockSpec((NC, ROW_BLK, 64), lambda i: (0, i, 0)),
            _col_spec(),
            _col_spec(),
            pl.BlockSpec((64, 128), lambda i: (0, 0)),
            pl.BlockSpec((64, 128), lambda i: (0, 0)),
            pl.BlockSpec((1, 128), lambda i: (0, 0)),
            pl.BlockSpec((128, 64), lambda i: (0, 0)),
        ],
        out_specs=pl.BlockSpec((ROW_BLK, 64), lambda i: (i, 0)),
        out_shape=jax.ShapeDtypeStruct((N_PAD, 64), jnp.float32),
    )(p1[0], p1[1], degd_col, degs_col, W1[:64], W1[64:], b1, W2)


def _fin(p2, degd_col, b2):
    return pl.pallas_call(
        _fin_body,
        grid=(N_PAD // ROW_BLK,),
        in_specs=[
            pl.BlockSpec((NC, ROW_BLK, 64), lambda i: (0, i, 0)),
            _col_spec(),
            pl.BlockSpec((1, 64), lambda i: (0, 0)),
        ],
        out_specs=pl.BlockSpec((ROW_BLK, 64), lambda i: (i, 0)),
        out_shape=jax.ShapeDtypeStruct((N_PAD, 64), jnp.float32),
    )(p2, degd_col, b2)


def kernel(x, edge_index, W1, b1, W2, b2):
    E = edge_index.shape[1]
    src = edge_index[0].astype(jnp.int32)
    dst = edge_index[1].astype(jnp.int32)

    fill2 = jnp.full((NW * EPT2 - 2 * E,), N_NODES, jnp.int32)
    comb = jnp.concatenate([src, dst + N_PAD, fill2]).reshape(NW, NCH2, CH)

    fill1 = jnp.full((NW * EPT1 - E,), N_NODES, jnp.int32)
    src3 = jnp.concatenate([src, fill1]).reshape(NW, NCH1, CH)
    dst3 = jnp.concatenate([dst, fill1]).reshape(NW, NCH1, CH)

    ones = jnp.ones((CH,), jnp.float32)
    zeros_deg = jnp.zeros((2 * N_PAD // NS,), jnp.float32)
    zeros64 = jnp.zeros((N_PAD // NS, 64), jnp.float32)

    dp = _deg_kernel(comb, ones, zeros_deg)
    deg = dp[0] + dp[1]
    degs_col = deg[:N_PAD].reshape(N_PAD, 1)
    degd_col = deg[N_PAD:].reshape(N_PAD, 1)

    x_pad = jnp.concatenate([x, jnp.zeros((N_PAD - N_NODES, 128), x.dtype)])
    a1 = _scale(x_pad, degs_col)
    a1s = jnp.stack([a1[:, :64], a1[:, 64:]])
    p1 = _agg2(a1s, src3, dst3, zeros64)
    a2 = _mid(p1, degd_col, degs_col, W1, b1.reshape(1, 128), W2)
    p2 = _agg1(a2.reshape(1, N_PAD, 64), src3, dst3, zeros64)
    z2 = _fin(p2[0], degd_col, b2.reshape(1, 64))
    return z2[:N_NODES]


# speedup vs baseline: 4.1451x; 1.0678x over previous
import functools

import jax
import jax.numpy as jnp
from jax import lax
from jax.experimental import pallas as pl
from jax.experimental.pallas import tpu as pltpu
from jax.experimental.pallas import tpu_sc as plsc

N_NODES = 10000
N_PAD = 10240
CH = 128
NC = 2
NS = 16
NW = NC * NS
NCH1 = 80
EPT1 = NCH1 * CH
NCH2 = 158
EPT2 = NCH2 * CH
ROW_BLK = 128

_mesh = plsc.VectorSubcoreMesh(core_axis_name="c", subcore_axis_name="s")
_sc_params = pltpu.CompilerParams(use_tc_tiling_on_sc=False)


@functools.partial(
    pl.kernel,
    out_type=jax.ShapeDtypeStruct((NC, 2 * N_PAD), jnp.float32),
    mesh=_mesh,
    compiler_params=_sc_params,
    scratch_types=[
        pltpu.VMEM((NCH2, CH), jnp.int32),
        pltpu.VMEM((CH,), jnp.float32),
        pltpu.VMEM_SHARED((2 * N_PAD,), jnp.float32),
    ],
)
def _deg_kernel(cidx_hbm, ones_hbm, zeros_hbm, out_hbm, idx_all, ones_v, acc):
    cc = lax.axis_index("c")
    ss = lax.axis_index("s")
    wid = ss * NC + cc
    span = 2 * N_PAD // NS
    pltpu.sync_copy(zeros_hbm, acc.at[pl.ds(ss * span, span)])
    pltpu.sync_copy(cidx_hbm.at[wid], idx_all)
    pltpu.sync_copy(ones_hbm, ones_v)
    plsc.subcore_barrier()

    @pl.loop(0, NCH2)
    def _(t):
        pltpu.sync_copy(ones_v, acc.at[idx_all.at[t]], add=True)

    plsc.subcore_barrier()
    pltpu.sync_copy(acc.at[pl.ds(ss * span, span)],
                    out_hbm.at[cc, pl.ds(ss * span, span)])


def _make_agg(phases):
    @functools.partial(
        pl.kernel,
        out_type=jax.ShapeDtypeStruct((phases, NC, N_PAD, 64), jnp.float32),
        mesh=_mesh,
        compiler_params=_sc_params,
        scratch_types=[
            pltpu.VMEM((NCH1, CH), jnp.int32),
            pltpu.VMEM((NCH1, CH), jnp.int32),
            [pltpu.VMEM((CH, 64), jnp.float32) for _ in range(4)],
            pltpu.VMEM_SHARED((N_PAD, 64), jnp.float32),
            [pltpu.SemaphoreType.DMA for _ in range(4)],
            [pltpu.SemaphoreType.DMA for _ in range(4)],
        ],
    )
    def agg(table_hbm, sidx_hbm, didx_hbm, zeros_hbm, out_hbm,
            sidx, didx, rows, acc, gsem, ssem):
        cc = lax.axis_index("c")
        ss = lax.axis_index("s")
        wid = ss * NC + cc
        rpt = N_PAD // NS
        pltpu.sync_copy(sidx_hbm.at[wid], sidx)
        pltpu.sync_copy(didx_hbm.at[wid], didx)

        def wait_gather(table, t, b):
            pltpu.make_async_copy(table.at[sidx.at[t]], rows[b], gsem[b]).wait()

        def wait_scatter(t, b):
            pltpu.make_async_copy(rows[b], acc.at[didx.at[t]], ssem[b]).wait()

        for ph in range(phases):
            pltpu.sync_copy(zeros_hbm, acc.at[pl.ds(ss * rpt, rpt)])
            plsc.subcore_barrier()
            table = table_hbm.at[ph]

            pltpu.async_copy(table.at[sidx.at[0]], rows[0], gsem[0])
            pltpu.async_copy(table.at[sidx.at[1]], rows[1], gsem[1])

            @pl.loop(0, NCH1 // 4)
            def _(g):
                t0 = 4 * g
                for b in range(4):
                    t = t0 + b
                    b2 = (b + 2) % 4
                    wait_gather(table, t, b)
                    pltpu.async_copy(rows[b], acc.at[didx.at[t]], ssem[b],
                                     add=True)
                    if b < 2:
                        @pl.when(g > 0)
                        def _():
                            wait_scatter(t - 2, b2)
                        pltpu.async_copy(table.at[sidx.at[t + 2]], rows[b2],
                                         gsem[b2])
                    else:
                        wait_scatter(t - 2, b2)

                        @pl.when(g < NCH1 // 4 - 1)
                        def _():
                            pltpu.async_copy(table.at[sidx.at[t + 2]],
                                             rows[b2], gsem[b2])

            wait_scatter(NCH1 - 2, 2)
            wait_scatter(NCH1 - 1, 3)
            plsc.subcore_barrier()
            pltpu.sync_copy(acc.at[pl.ds(ss * rpt, rpt)],
                            out_hbm.at[ph, cc, pl.ds(ss * rpt, rpt)])

    return agg


_agg2 = _make_agg(2)
_agg1 = _make_agg(1)


def _scale_body(x_ref, ds_ref, o_ref):
    ns = lax.rsqrt(jnp.maximum(ds_ref[...], 1.0))
    o_ref[...] = x_ref[...] * ns


def _mid_body(lo_ref, hi_ref, dd_ref, ds_ref, w1lo_ref, w1hi_ref, b1_ref,
              w2_ref, o_ref):
    nd = lax.rsqrt(jnp.maximum(dd_ref[...], 1.0))
    ns = lax.rsqrt(jnp.maximum(ds_ref[...], 1.0))
    agg_lo = (lo_ref[0] + lo_ref[1]) * nd
    agg_hi = (hi_ref[0] + hi_ref[1]) * nd
    z1 = (jnp.dot(agg_lo, w1lo_ref[...], preferred_element_type=jnp.float32)
          + jnp.dot(agg_hi, w1hi_ref[...], preferred_element_type=jnp.float32)
          + b1_ref[...])
    o_ref[...] = jnp.dot(z1 * ns, w2_ref[...], preferred_element_type=jnp.float32)


def _fin_body(p_ref, dd_ref, b2_ref, o_ref):
    nd = lax.rsqrt(jnp.maximum(dd_ref[...], 1.0))
    o_ref[...] = (p_ref[0] + p_ref[1]) * nd + b2_ref[...]


def _col_spec():
    return pl.BlockSpec((ROW_BLK, 1), lambda i: (i, 0))


def _scale(x_pad, degs_col):
    return pl.pallas_call(
        _scale_body,
        grid=(N_PAD // ROW_BLK,),
        in_specs=[pl.BlockSpec((ROW_BLK, 128), lambda i: (i, 0)), _col_spec()],
        out_specs=pl.BlockSpec((ROW_BLK, 128), lambda i: (i, 0)),
        out_shape=jax.ShapeDtypeStruct((N_PAD, 128), jnp.float32),
    )(x_pad, degs_col)


def _mid(p1, degd_col, degs_col, W1, b1, W2):
    return pl.pallas_call(
        _mid_body,
        grid=(N_PAD // ROW_BLK,),
        in_specs=[
            pl.BlockSpec((NC, ROW_BLK, 64), lambda i: (0, i, 0)),
            pl.BlockSpec((NC, ROW_BLK, 64), lambda i: (0, i, 0)),
            _col_spec(),
            _col_spec(),
            pl.BlockSpec((64, 128), lambda i: (0, 0)),
            pl.BlockSpec((64, 128), lambda i: (0, 0)),
            pl.BlockSpec((1, 128), lambda i: (0, 0)),
            pl.BlockSpec((128, 64), lambda i: (0, 0)),
        ],
        out_specs=pl.BlockSpec((ROW_BLK, 64), lambda i: (i, 0)),
        out_shape=jax.ShapeDtypeStruct((N_PAD, 64), jnp.float32),
    )(p1[0], p1[1], degd_col, degs_col, W1[:64], W1[64:], b1, W2)


def _fin(p2, degd_col, b2):
    return pl.pallas_call(
        _fin_body,
        grid=(N_PAD // ROW_BLK,),
        in_specs=[
            pl.BlockSpec((NC, ROW_BLK, 64), lambda i: (0, i, 0)),
            _col_spec(),
            pl.BlockSpec((1, 64), lambda i: (0, 0)),
        ],
        out_specs=pl.BlockSpec((ROW_BLK, 64), lambda i: (i, 0)),
        out_shape=jax.ShapeDtypeStruct((N_PAD, 64), jnp.float32),
    )(p2, degd_col, b2)


def kernel(x, edge_index, W1, b1, W2, b2):
    E = edge_index.shape[1]
    src = edge_index[0].astype(jnp.int32)
    dst = edge_index[1].astype(jnp.int32)

    fill2 = jnp.full((NW * EPT2 - 2 * E,), N_NODES, jnp.int32)
    comb = jnp.concatenate([src, dst + N_PAD, fill2]).reshape(NW, NCH2, CH)

    fill1 = jnp.full((NW * EPT1 - E,), N_NODES, jnp.int32)
    src3 = jnp.concatenate([src, fill1]).reshape(NW, NCH1, CH)
    dst3 = jnp.concatenate([dst, fill1]).reshape(NW, NCH1, CH)

    ones = jnp.ones((CH,), jnp.float32)
    zeros_deg = jnp.zeros((2 * N_PAD // NS,), jnp.float32)
    zeros64 = jnp.zeros((N_PAD // NS, 64), jnp.float32)

    dp = _deg_kernel(comb, ones, zeros_deg)
    deg = dp[0] + dp[1]
    degs_col = deg[:N_PAD].reshape(N_PAD, 1)
    degd_col = deg[N_PAD:].reshape(N_PAD, 1)

    x_pad = jnp.concatenate([x, jnp.zeros((N_PAD - N_NODES, 128), x.dtype)])
    a1 = _scale(x_pad, degs_col)
    a1s = jnp.stack([a1[:, :64], a1[:, 64:]])
    p1 = _agg2(a1s, src3, dst3, zeros64)
    a2 = _mid(p1, degd_col, degs_col, W1, b1.reshape(1, 128), W2)
    p2 = _agg1(a2.reshape(1, N_PAD, 64), src3, dst3, zeros64)
    z2 = _fin(p2[0], degd_col, b2.reshape(1, 64))
    return z2[:N_NODES]
